# baseline (speedup 1.0000x reference)
import functools

import jax
import jax.numpy as jnp
from jax import lax
from jax.experimental import pallas as pl
from jax.experimental.pallas import tpu as pltpu
from jax.experimental.pallas import tpu_sc as plsc

N = 10000
E = 320000
R = 7
SEG = R * N
NG = 32

NTEC = 16
NSC = 2
B = 80
ROWS = SEG // NTEC
ZR = 125
NZ = ROWS // ZR

RNG = 8960
NRANGE = 8
CAP = 43008
E_PAD = NRANGE * CAP
BE = 48
KK = 2
NBT = 56
NGRP2 = NBT // KK
ACC_R = RNG + 16
ZROW = 51
JUNK = RNG

BN_BLK = 400
NBLK = N // BN_BLK


def _sc_mesh():
    return plsc.VectorSubcoreMesh(core_axis_name="c", subcore_axis_name="s")


def _make_agg_call(fdim):
    out_t = jax.ShapeDtypeStruct((SEG, fdim, 128), jnp.float32)
    scratch = [
        pltpu.VMEM((2, KK, 2, BE), jnp.int32),
        pltpu.VMEM((2, KK, BE, 128), jnp.float32),
        pltpu.VMEM((ZROW, 128), jnp.float32),
        pltpu.VMEM_SHARED((ACC_R, 128), jnp.float32),
        pltpu.SemaphoreType.DMA,
        pltpu.SemaphoreType.DMA,
        pltpu.SemaphoreType.DMA,
        pltpu.SemaphoreType.DMA,
        pltpu.SemaphoreType.DMA,
    ]

    @functools.partial(pl.kernel, out_type=out_t, mesh=_sc_mesh(),
                       scratch_types=scratch,
                       compiler_params=pltpu.CompilerParams(
                           use_tc_tiling_on_sc=False))
    def agg_kernel(h4, ij, z51, agg, ijbuf, gbuf, zbuf, acc_sh,
                   isem, gsem, ssem0, ssem1, fsem):
        cid = lax.axis_index("c")
        sid = lax.axis_index("s")
        pltpu.async_copy(z51, zbuf, isem).wait()

        def drain_scatters(s, ssem):
            for _ in range(KK):
                pltpu.make_async_copy(gbuf.at[s, 0],
                                      acc_sh.at[pl.ds(0, BE)], ssem).wait()

        def do_pass(ri, fc):
            zc = [pltpu.async_copy(zbuf,
                                   acc_sh.at[pl.ds(sid * 561 + ZROW * z, ZROW)],
                                   fsem) for z in range(11)]
            for c in zc:
                c.wait()
            plsc.subcore_barrier()
            base = ri * CAP

            def do_group(g, s, ssem, drain_prev):
                if drain_prev:
                    drain_scatters(s, ssem)
                ic = []
                for k in range(KK):
                    off = base + ((g * KK + k) * NTEC + sid) * BE
                    ic.append(pltpu.async_copy(ij.at[fc, :, pl.ds(off, BE)],
                                               ijbuf.at[s, k], isem))
                for c in ic:
                    c.wait()
                gc = [pltpu.async_copy(h4.at[ijbuf.at[s, k, 0]],
                                       gbuf.at[s, k], gsem)
                      for k in range(KK)]
                for c in gc:
                    c.wait()
                for k in range(KK):
                    pltpu.async_copy(gbuf.at[s, k],
                                     acc_sh.at[ijbuf.at[s, k, 1]],
                                     ssem, add=True)

            do_group(0, 0, ssem0, False)
            do_group(1, 1, ssem1, False)

            @pl.loop(2, NGRP2, step=2)
            def _grp(go):
                do_group(go, 0, ssem0, True)
                do_group(go + 1, 1, ssem1, True)

            drain_scatters(0, ssem0)
            drain_scatters(1, ssem1)
            plsc.subcore_barrier()

            @pl.when(ri != NRANGE - 1)
            def _flush_full():
                pltpu.async_copy(
                    acc_sh.at[pl.ds(sid * 560, 560)],
                    agg.at[pl.ds(ri * RNG + sid * 560, 560), fc], fsem).wait()

            @pl.when(ri == NRANGE - 1)
            def _flush_tail():
                pltpu.async_copy(
                    acc_sh.at[pl.ds(sid * 455, 455)],
                    agg.at[pl.ds(ri * RNG + sid * 455, 455), fc], fsem).wait()

            plsc.subcore_barrier()

        if fdim == 1:
            @pl.loop(0, NRANGE // NSC)
            def _ri(i):
                do_pass(cid * (NRANGE // NSC) + i, 0)
        else:
            @pl.loop(0, NRANGE)
            def _ri(ri):
                for p in range(fdim // NSC):
                    do_pass(ri, cid * (fdim // NSC) + p)

    return agg_kernel


EPT_D = E // (NSC * NTEC)
NB_D = EPT_D // B


def _deg_call():
    out_t = jax.ShapeDtypeStruct((NSC, SEG, 16), jnp.float32)
    scratch = [
        pltpu.VMEM((NB_D, B), jnp.int32),
        pltpu.VMEM((B, 16), jnp.float32),
        pltpu.VMEM((ZR, 16), jnp.float32),
        pltpu.VMEM_SHARED((SEG, 16), jnp.float32),
        pltpu.SemaphoreType.DMA,
        pltpu.SemaphoreType.DMA,
    ]

    @functools.partial(pl.kernel, out_type=out_t, mesh=_sc_mesh(),
                       scratch_types=scratch,
                       compiler_params=pltpu.CompilerParams(
                           use_tc_tiling_on_sc=False))
    def deg_kernel(combr, deg, comb_v, ones_v, zbuf, acc_sh, isem, fsem):
        cid = lax.axis_index("c")
        sid = lax.axis_index("s")
        tile = cid * NTEC + sid
        t0 = sid * ROWS
        pltpu.async_copy(combr.at[tile], comb_v, isem).wait()

        @pl.loop(0, B)
        def _fill_ones(i):
            ones_v[i] = jnp.full((16,), 1.0, jnp.float32)

        @pl.loop(0, ZR)
        def _fill_zeros(i):
            zbuf[i] = jnp.zeros((16,), jnp.float32)

        zc = [pltpu.async_copy(zbuf, acc_sh.at[pl.ds(t0 + ZR * z, ZR)], fsem)
              for z in range(NZ)]
        for c in zc:
            c.wait()
        plsc.subcore_barrier()

        @pl.loop(0, NB_D)
        def _blk(j):
            pltpu.sync_copy(ones_v, acc_sh.at[comb_v.at[j]], add=True)

        plsc.subcore_barrier()
        pltpu.async_copy(acc_sh.at[pl.ds(t0, ROWS)],
                         deg.at[cid, pl.ds(t0, ROWS)], fsem).wait()

    return deg_kernel


def _mm_kernel(h_ref, agg_ref, deg_ref, wrel_ref, wroot_ref, b_ref,
               g_ref, be_ref, m_ref, v_ref, o_ref, *, relu):
    acc = jnp.dot(h_ref[...], wroot_ref[...],
                  preferred_element_type=jnp.float32)
    deg = deg_ref[0] + deg_ref[1]
    for r in range(R):
        inv = 1.0 / jnp.maximum(deg[r][:, :1], 1.0)
        acc = acc + jnp.dot(agg_ref[r] * inv, wrel_ref[r],
                            preferred_element_type=jnp.float32)
    acc = acc + b_ref[...]
    acc = (acc - m_ref[...]) * (g_ref[...] * lax.rsqrt(v_ref[...] + 1e-5))
    acc = acc + be_ref[...]
    if relu:
        acc = jnp.maximum(acc, 0.0)
    o_ref[...] = acc


def _mm_call(h, agg, deg4, wrel, wroot, b, g, be, m, v, relu):
    din = h.shape[1]
    vspec = pl.BlockSpec((1, 512), lambda i: (0, 0))
    return pl.pallas_call(
        functools.partial(_mm_kernel, relu=relu),
        grid=(NBLK,),
        in_specs=[
            pl.BlockSpec((BN_BLK, din), lambda i: (i, 0)),
            pl.BlockSpec((R, BN_BLK, din), lambda i: (0, i, 0)),
            pl.BlockSpec((NSC, R, BN_BLK, 16), lambda i: (0, 0, i, 0)),
            pl.BlockSpec((R, din, 512), lambda i: (0, 0, 0)),
            pl.BlockSpec((din, 512), lambda i: (0, 0)),
            vspec, vspec, vspec, vspec, vspec,
        ],
        out_specs=pl.BlockSpec((BN_BLK, 512), lambda i: (i, 0)),
        out_shape=jax.ShapeDtypeStruct((N, 512), jnp.float32),
    )(h, agg, deg4, wrel, wroot, b, g, be, m, v)


def _pool_kernel(h_ref, batch_ref, w1_ref, b1_ref, w2_ref, b2_ref, o_ref,
                 sums_ref, cnt_ref):
    i = pl.program_id(0)

    @pl.when(i == 0)
    def _init():
        sums_ref[...] = jnp.zeros_like(sums_ref)
        cnt_ref[...] = jnp.zeros_like(cnt_ref)

    bb = batch_ref[0, 0, :]
    oh = (bb[:, None] == lax.broadcasted_iota(jnp.int32, (BN_BLK, NG), 1)
          ).astype(jnp.float32)
    dn = (((0,), (0,)), ((), ()))
    sums_ref[...] += lax.dot_general(oh, h_ref[...], dn,
                                     preferred_element_type=jnp.float32)
    cnt_ref[...] += lax.dot_general(oh, jnp.ones((BN_BLK, 8), jnp.float32),
                                    dn, preferred_element_type=jnp.float32)

    @pl.when(i == NBLK - 1)
    def _final():
        inv = 1.0 / jnp.maximum(cnt_ref[:, :1], 1.0)
        pooled = sums_ref[...] * inv
        hid = jnp.dot(pooled, w1_ref[...],
                      preferred_element_type=jnp.float32) + b1_ref[...]
        hid = jnp.maximum(hid, 0.0)
        o_ref[...] = jnp.dot(hid, w2_ref[...],
                             preferred_element_type=jnp.float32) + b2_ref[...]


def _pool_call(h, batch3, w1, b1, w2, b2):
    return pl.pallas_call(
        _pool_kernel,
        grid=(NBLK,),
        in_specs=[
            pl.BlockSpec((BN_BLK, 512), lambda i: (i, 0)),
            pl.BlockSpec((1, 1, BN_BLK), lambda i: (i, 0, 0)),
            pl.BlockSpec((512, 300), lambda i: (0, 0)),
            pl.BlockSpec((1, 300), lambda i: (0, 0)),
            pl.BlockSpec((300, 300), lambda i: (0, 0)),
            pl.BlockSpec((1, 300), lambda i: (0, 0)),
        ],
        out_specs=pl.BlockSpec((NG, 300), lambda i: (0, 0)),
        out_shape=jax.ShapeDtypeStruct((NG, 300), jnp.float32),
        scratch_shapes=[
            pltpu.VMEM((NG, 512), jnp.float32),
            pltpu.VMEM((NG, 8), jnp.float32),
        ],
    )(h, batch3, w1, b1, w2, b2)


_agg4 = _make_agg_call(4)
_agg1 = _make_agg_call(1)
_deg = _deg_call()


def _sorted_edge_indices(src, comb):
    order = jnp.argsort(comb)
    comb_s = comb[order]
    src_s = src[order]
    rid = comb_s // RNG
    bounds = jnp.searchsorted(
        comb_s, (jnp.arange(NRANGE) * RNG).astype(jnp.int32)).astype(jnp.int32)
    rank = jnp.arange(E, dtype=jnp.int32) - bounds[rid]
    pos = rid * CAP + rank
    gsrc = jnp.zeros((E_PAD,), jnp.int32).at[pos].set(src_s, mode='drop')
    loc = jnp.full((E_PAD,), JUNK, jnp.int32).at[pos].set(
        comb_s - rid * RNG, mode='drop')
    ij4 = jnp.stack([gsrc[None, :] * 4 + jnp.arange(4, dtype=jnp.int32)[:, None],
                     jnp.broadcast_to(loc, (4, E_PAD))], axis=1)
    ij1 = jnp.stack([gsrc[None, :], loc[None, :]], axis=1)
    return ij4, ij1


def kernel(x, edge_index, edge_type, batch, W_rel1, W_root1, b1, W_rel2,
           W_root2, b2, W_rel3, W_root3, b3, W_rel4, W_root4, b4, W_rel5,
           W_root5, b5, W_rel6, W_root6, b6, bn_gamma, bn_beta, bn_mean,
           bn_var, ph_w1, ph_b1, ph_w2, ph_b2):
    src = edge_index[0]
    dst = edge_index[1]
    comb = edge_type * N + dst
    combd = comb.reshape(NSC * NTEC, NB_D, B)
    ij4, ij1 = _sorted_edge_indices(src, comb)
    z51 = jnp.zeros((ZROW, 128), jnp.float32)

    deg = _deg(combd)
    deg4 = deg.reshape(NSC, R, N, 16)

    g = bn_gamma.reshape(1, 512)
    be = bn_beta.reshape(1, 512)
    m = bn_mean.reshape(1, 512)
    v = bn_var.reshape(1, 512)

    x_pad = jnp.pad(x, ((0, 0), (0, 106)))
    w1_pad = jnp.pad(W_rel1, ((0, 0), (0, 106), (0, 0)))
    wr1_pad = jnp.pad(W_root1, ((0, 106), (0, 0)))

    layers = [
        (ij1, _agg1, 1, w1_pad, wr1_pad, b1),
        (ij4, _agg4, 4, W_rel2, W_root2, b2),
        (ij4, _agg4, 4, W_rel3, W_root3, b3),
        (ij4, _agg4, 4, W_rel4, W_root4, b4),
        (ij4, _agg4, 4, W_rel5, W_root5, b5),
        (ij4, _agg4, 4, W_rel6, W_root6, b6),
    ]

    h = x_pad
    for li, (ij, aggf, fdim, wrel, wroot, bb) in enumerate(layers):
        h4 = h.reshape(N * fdim, 128)
        agg = aggf(h4, ij, z51)
        agg_r = agg.reshape(R, N, fdim * 128)
        h = _mm_call(h, agg_r, deg4, wrel, wroot, bb.reshape(1, 512),
                     g, be, m, v, relu=(li < 5))

    return _pool_call(h, batch.reshape(NBLK, 1, BN_BLK), ph_w1,
                      ph_b1.reshape(1, 300), ph_w2, ph_b2.reshape(1, 300))


# speedup vs baseline: 1.0426x; 1.0426x over previous
import functools

import jax
import jax.numpy as jnp
from jax import lax
from jax.experimental import pallas as pl
from jax.experimental.pallas import tpu as pltpu
from jax.experimental.pallas import tpu_sc as plsc

N = 10000
E = 320000
R = 7
SEG = R * N
NG = 32

NTEC = 16
NSC = 2
B = 80
ROWS = SEG // NTEC
ZR = 125
NZ = ROWS // ZR

RNG = 8960
NRANGE = 8
CAP = 43008
E_PAD = NRANGE * CAP
BE = 48
KK = 2
NBT = 56
NGRP2 = NBT // KK
ACC_R = RNG + 16
ZROW = 51
JUNK = RNG

BN_BLK = 400
NBLK = N // BN_BLK


def _sc_mesh():
    return plsc.VectorSubcoreMesh(core_axis_name="c", subcore_axis_name="s")


def _make_agg_call(fdim):
    out_t = jax.ShapeDtypeStruct((SEG, fdim, 128), jnp.float32)
    scratch = [
        pltpu.VMEM((2, KK, 2, BE), jnp.int32),
        pltpu.VMEM((2, KK, BE, 128), jnp.float32),
        pltpu.VMEM((ZROW, 128), jnp.float32),
        pltpu.VMEM_SHARED((ACC_R, 128), jnp.float32),
        pltpu.SemaphoreType.DMA,
        pltpu.SemaphoreType.DMA,
        pltpu.SemaphoreType.DMA,
        pltpu.SemaphoreType.DMA,
        pltpu.SemaphoreType.DMA,
    ]

    @functools.partial(pl.kernel, out_type=out_t, mesh=_sc_mesh(),
                       scratch_types=scratch,
                       compiler_params=pltpu.CompilerParams(
                           use_tc_tiling_on_sc=False))
    def agg_kernel(h4, ij, z51, agg, ijbuf, gbuf, zbuf, acc_sh,
                   isem, gsem, ssem0, ssem1, fsem):
        cid = lax.axis_index("c")
        sid = lax.axis_index("s")
        pltpu.async_copy(z51, zbuf, isem).wait()

        def drain_scatters(s, ssem):
            for _ in range(KK):
                pltpu.make_async_copy(gbuf.at[s, 0],
                                      acc_sh.at[pl.ds(0, BE)], ssem).wait()

        def do_pass(ri, fc):
            zc = [pltpu.async_copy(zbuf,
                                   acc_sh.at[pl.ds(sid * 561 + ZROW * z, ZROW)],
                                   fsem) for z in range(11)]
            for c in zc:
                c.wait()
            plsc.subcore_barrier()
            base = ri * CAP

            def do_group(g, s, ssem, drain_prev):
                if drain_prev:
                    drain_scatters(s, ssem)
                ic = []
                for k in range(KK):
                    off = base + (sid * NBT + g * KK + k) * BE
                    ic.append(pltpu.async_copy(ij.at[fc, :, pl.ds(off, BE)],
                                               ijbuf.at[s, k], isem))
                for c in ic:
                    c.wait()
                gc = [pltpu.async_copy(h4.at[ijbuf.at[s, k, 0]],
                                       gbuf.at[s, k], gsem)
                      for k in range(KK)]
                for c in gc:
                    c.wait()
                for k in range(KK):
                    pltpu.async_copy(gbuf.at[s, k],
                                     acc_sh.at[ijbuf.at[s, k, 1]],
                                     ssem, add=True)

            do_group(0, 0, ssem0, False)
            do_group(1, 1, ssem1, False)

            @pl.loop(2, NGRP2, step=2)
            def _grp(go):
                do_group(go, 0, ssem0, True)
                do_group(go + 1, 1, ssem1, True)

            drain_scatters(0, ssem0)
            drain_scatters(1, ssem1)
            plsc.subcore_barrier()

            @pl.when(ri != NRANGE - 1)
            def _flush_full():
                pltpu.async_copy(
                    acc_sh.at[pl.ds(sid * 560, 560)],
                    agg.at[pl.ds(ri * RNG + sid * 560, 560), fc], fsem).wait()

            @pl.when(ri == NRANGE - 1)
            def _flush_tail():
                pltpu.async_copy(
                    acc_sh.at[pl.ds(sid * 455, 455)],
                    agg.at[pl.ds(ri * RNG + sid * 455, 455), fc], fsem).wait()

            plsc.subcore_barrier()

        if fdim == 1:
            @pl.loop(0, NRANGE // NSC)
            def _ri(i):
                do_pass(cid * (NRANGE // NSC) + i, 0)
        else:
            @pl.loop(0, NRANGE)
            def _ri(ri):
                for p in range(fdim // NSC):
                    do_pass(ri, cid * (fdim // NSC) + p)

    return agg_kernel


EPT_D = E // (NSC * NTEC)
NB_D = EPT_D // B


def _deg_call():
    out_t = jax.ShapeDtypeStruct((NSC, SEG, 16), jnp.float32)
    scratch = [
        pltpu.VMEM((NB_D, B), jnp.int32),
        pltpu.VMEM((B, 16), jnp.float32),
        pltpu.VMEM((ZR, 16), jnp.float32),
        pltpu.VMEM_SHARED((SEG, 16), jnp.float32),
        pltpu.SemaphoreType.DMA,
        pltpu.SemaphoreType.DMA,
    ]

    @functools.partial(pl.kernel, out_type=out_t, mesh=_sc_mesh(),
                       scratch_types=scratch,
                       compiler_params=pltpu.CompilerParams(
                           use_tc_tiling_on_sc=False))
    def deg_kernel(combr, deg, comb_v, ones_v, zbuf, acc_sh, isem, fsem):
        cid = lax.axis_index("c")
        sid = lax.axis_index("s")
        tile = cid * NTEC + sid
        t0 = sid * ROWS
        pltpu.async_copy(combr.at[tile], comb_v, isem).wait()

        @pl.loop(0, B)
        def _fill_ones(i):
            ones_v[i] = jnp.full((16,), 1.0, jnp.float32)

        @pl.loop(0, ZR)
        def _fill_zeros(i):
            zbuf[i] = jnp.zeros((16,), jnp.float32)

        zc = [pltpu.async_copy(zbuf, acc_sh.at[pl.ds(t0 + ZR * z, ZR)], fsem)
              for z in range(NZ)]
        for c in zc:
            c.wait()
        plsc.subcore_barrier()

        @pl.loop(0, NB_D)
        def _blk(j):
            pltpu.sync_copy(ones_v, acc_sh.at[comb_v.at[j]], add=True)

        plsc.subcore_barrier()
        pltpu.async_copy(acc_sh.at[pl.ds(t0, ROWS)],
                         deg.at[cid, pl.ds(t0, ROWS)], fsem).wait()

    return deg_kernel


def _mm_kernel(h_ref, agg_ref, deg_ref, wrel_ref, wroot_ref, b_ref,
               g_ref, be_ref, m_ref, v_ref, o_ref, *, relu):
    acc = jnp.dot(h_ref[...], wroot_ref[...],
                  preferred_element_type=jnp.float32)
    deg = deg_ref[0] + deg_ref[1]
    for r in range(R):
        inv = 1.0 / jnp.maximum(deg[r][:, :1], 1.0)
        acc = acc + jnp.dot(agg_ref[r] * inv, wrel_ref[r],
                            preferred_element_type=jnp.float32)
    acc = acc + b_ref[...]
    acc = (acc - m_ref[...]) * (g_ref[...] * lax.rsqrt(v_ref[...] + 1e-5))
    acc = acc + be_ref[...]
    if relu:
        acc = jnp.maximum(acc, 0.0)
    o_ref[...] = acc


def _mm_call(h, agg, deg4, wrel, wroot, b, g, be, m, v, relu):
    din = h.shape[1]
    vspec = pl.BlockSpec((1, 512), lambda i: (0, 0))
    return pl.pallas_call(
        functools.partial(_mm_kernel, relu=relu),
        grid=(NBLK,),
        in_specs=[
            pl.BlockSpec((BN_BLK, din), lambda i: (i, 0)),
            pl.BlockSpec((R, BN_BLK, din), lambda i: (0, i, 0)),
            pl.BlockSpec((NSC, R, BN_BLK, 16), lambda i: (0, 0, i, 0)),
            pl.BlockSpec((R, din, 512), lambda i: (0, 0, 0)),
            pl.BlockSpec((din, 512), lambda i: (0, 0)),
            vspec, vspec, vspec, vspec, vspec,
        ],
        out_specs=pl.BlockSpec((BN_BLK, 512), lambda i: (i, 0)),
        out_shape=jax.ShapeDtypeStruct((N, 512), jnp.float32),
    )(h, agg, deg4, wrel, wroot, b, g, be, m, v)


def _pool_kernel(h_ref, batch_ref, w1_ref, b1_ref, w2_ref, b2_ref, o_ref,
                 sums_ref, cnt_ref):
    i = pl.program_id(0)

    @pl.when(i == 0)
    def _init():
        sums_ref[...] = jnp.zeros_like(sums_ref)
        cnt_ref[...] = jnp.zeros_like(cnt_ref)

    bb = batch_ref[0, 0, :]
    oh = (bb[:, None] == lax.broadcasted_iota(jnp.int32, (BN_BLK, NG), 1)
          ).astype(jnp.float32)
    dn = (((0,), (0,)), ((), ()))
    sums_ref[...] += lax.dot_general(oh, h_ref[...], dn,
                                     preferred_element_type=jnp.float32)
    cnt_ref[...] += lax.dot_general(oh, jnp.ones((BN_BLK, 8), jnp.float32),
                                    dn, preferred_element_type=jnp.float32)

    @pl.when(i == NBLK - 1)
    def _final():
        inv = 1.0 / jnp.maximum(cnt_ref[:, :1], 1.0)
        pooled = sums_ref[...] * inv
        hid = jnp.dot(pooled, w1_ref[...],
                      preferred_element_type=jnp.float32) + b1_ref[...]
        hid = jnp.maximum(hid, 0.0)
        o_ref[...] = jnp.dot(hid, w2_ref[...],
                             preferred_element_type=jnp.float32) + b2_ref[...]


def _pool_call(h, batch3, w1, b1, w2, b2):
    return pl.pallas_call(
        _pool_kernel,
        grid=(NBLK,),
        in_specs=[
            pl.BlockSpec((BN_BLK, 512), lambda i: (i, 0)),
            pl.BlockSpec((1, 1, BN_BLK), lambda i: (i, 0, 0)),
            pl.BlockSpec((512, 300), lambda i: (0, 0)),
            pl.BlockSpec((1, 300), lambda i: (0, 0)),
            pl.BlockSpec((300, 300), lambda i: (0, 0)),
            pl.BlockSpec((1, 300), lambda i: (0, 0)),
        ],
        out_specs=pl.BlockSpec((NG, 300), lambda i: (0, 0)),
        out_shape=jax.ShapeDtypeStruct((NG, 300), jnp.float32),
        scratch_shapes=[
            pltpu.VMEM((NG, 512), jnp.float32),
            pltpu.VMEM((NG, 8), jnp.float32),
        ],
    )(h, batch3, w1, b1, w2, b2)


_agg4 = _make_agg_call(4)
_agg1 = _make_agg_call(1)
_deg = _deg_call()


def _sorted_edge_indices(src, comb):
    order = jnp.argsort(comb)
    comb_s = comb[order]
    src_s = src[order]
    rid = comb_s // RNG
    bounds = jnp.searchsorted(
        comb_s, (jnp.arange(NRANGE) * RNG).astype(jnp.int32)).astype(jnp.int32)
    rank = jnp.arange(E, dtype=jnp.int32) - bounds[rid]
    pos = rid * CAP + rank
    gsrc = jnp.zeros((E_PAD,), jnp.int32).at[pos].set(src_s, mode='drop')
    loc = jnp.full((E_PAD,), JUNK, jnp.int32).at[pos].set(
        comb_s - rid * RNG, mode='drop')
    ij4 = jnp.stack([gsrc[None, :] * 4 + jnp.arange(4, dtype=jnp.int32)[:, None],
                     jnp.broadcast_to(loc, (4, E_PAD))], axis=1)
    ij1 = jnp.stack([gsrc[None, :], loc[None, :]], axis=1)
    return ij4, ij1


def kernel(x, edge_index, edge_type, batch, W_rel1, W_root1, b1, W_rel2,
           W_root2, b2, W_rel3, W_root3, b3, W_rel4, W_root4, b4, W_rel5,
           W_root5, b5, W_rel6, W_root6, b6, bn_gamma, bn_beta, bn_mean,
           bn_var, ph_w1, ph_b1, ph_w2, ph_b2):
    src = edge_index[0]
    dst = edge_index[1]
    comb = edge_type * N + dst
    combd = comb.reshape(NSC * NTEC, NB_D, B)
    ij4, ij1 = _sorted_edge_indices(src, comb)
    z51 = jnp.zeros((ZROW, 128), jnp.float32)

    deg = _deg(combd)
    deg4 = deg.reshape(NSC, R, N, 16)

    g = bn_gamma.reshape(1, 512)
    be = bn_beta.reshape(1, 512)
    m = bn_mean.reshape(1, 512)
    v = bn_var.reshape(1, 512)

    x_pad = jnp.pad(x, ((0, 0), (0, 106)))
    w1_pad = jnp.pad(W_rel1, ((0, 0), (0, 106), (0, 0)))
    wr1_pad = jnp.pad(W_root1, ((0, 106), (0, 0)))

    layers = [
        (ij1, _agg1, 1, w1_pad, wr1_pad, b1),
        (ij4, _agg4, 4, W_rel2, W_root2, b2),
        (ij4, _agg4, 4, W_rel3, W_root3, b3),
        (ij4, _agg4, 4, W_rel4, W_root4, b4),
        (ij4, _agg4, 4, W_rel5, W_root5, b5),
        (ij4, _agg4, 4, W_rel6, W_root6, b6),
    ]

    h = x_pad
    for li, (ij, aggf, fdim, wrel, wroot, bb) in enumerate(layers):
        h4 = h.reshape(N * fdim, 128)
        agg = aggf(h4, ij, z51)
        agg_r = agg.reshape(R, N, fdim * 128)
        h = _mm_call(h, agg_r, deg4, wrel, wroot, bb.reshape(1, 512),
                     g, be, m, v, relu=(li < 5))

    return _pool_call(h, batch.reshape(NBLK, 1, BN_BLK), ph_w1,
                      ph_b1.reshape(1, 300), ph_w2, ph_b2.reshape(1, 300))


# speedup vs baseline: 2.0362x; 1.9530x over previous
import functools

import jax
import jax.numpy as jnp
from jax import lax
from jax.experimental import pallas as pl
from jax.experimental.pallas import tpu as pltpu
from jax.experimental.pallas import tpu_sc as plsc

N = 10000
E = 320000
R = 7
SEG = R * N
NG = 32

NTEC = 16
NSC = 2
EPT = E // NTEC
B = 80
NB = EPT // B
K = 5
NGRP = NB // K
ROWS = SEG // NTEC
ZR = 125
NZ = ROWS // ZR

BN_BLK = 400
NBLK = N // BN_BLK


def _sc_mesh():
    return plsc.VectorSubcoreMesh(core_axis_name="c", subcore_axis_name="s")


def _make_agg_call(nchunk_per_sc):
    fdim = nchunk_per_sc * NSC
    out_t = jax.ShapeDtypeStruct((SEG, fdim, 16), jnp.float32)
    scratch = [
        pltpu.VMEM((NB, B), jnp.int32),
        pltpu.VMEM((2, K, B), jnp.int32),
        pltpu.VMEM((2, K, B, 16), jnp.float32),
        pltpu.VMEM((ZR, 16), jnp.float32),
        pltpu.VMEM_SHARED((SEG, 16), jnp.float32),
        pltpu.SemaphoreType.DMA,
        pltpu.SemaphoreType.DMA,
        pltpu.SemaphoreType.DMA,
        pltpu.SemaphoreType.DMA,
        pltpu.SemaphoreType.DMA,
    ]

    @functools.partial(pl.kernel, out_type=out_t, mesh=_sc_mesh(),
                       scratch_types=scratch,
                       compiler_params=pltpu.CompilerParams(
                           use_tc_tiling_on_sc=False))
    def agg_kernel(h2, srcall, combr, agg, comb_v, sidx, gbuf, zbuf, acc_sh,
                   isem, gsem, ssem0, ssem1, fsem):
        cid = lax.axis_index("c")
        sid = lax.axis_index("s")
        t0 = sid * ROWS
        pltpu.async_copy(combr.at[sid], comb_v, isem).wait()

        @pl.loop(0, ZR)
        def _fill_zeros(i):
            zbuf[i] = jnp.zeros((16,), jnp.float32)

        def drain_scatters(s, ssem):
            for _ in range(K):
                pltpu.make_async_copy(gbuf.at[s, 0],
                                      acc_sh.at[pl.ds(0, B)], ssem).wait()

        def do_group(g, s, ssem, row, drain_prev):
            if drain_prev:
                drain_scatters(s, ssem)
            pltpu.async_copy(srcall.at[row, g], sidx.at[s], isem).wait()
            gc = [pltpu.async_copy(h2.at[sidx.at[s, k]], gbuf.at[s, k], gsem)
                  for k in range(K)]
            for c in gc:
                c.wait()
            for k in range(K):
                pltpu.async_copy(gbuf.at[s, k], acc_sh.at[comb_v.at[g * K + k]],
                                 ssem, add=True)

        @pl.loop(0, nchunk_per_sc)
        def _chunk(cc):
            f = cid * nchunk_per_sc + cc
            row = f * 16 + sid
            zc = [pltpu.async_copy(zbuf, acc_sh.at[pl.ds(t0 + ZR * z, ZR)],
                                   fsem) for z in range(NZ)]
            for c in zc:
                c.wait()
            plsc.subcore_barrier()
            do_group(0, 0, ssem0, row, False)
            do_group(1, 1, ssem1, row, False)

            @pl.loop(2, NGRP, step=2)
            def _grp(go):
                do_group(go, 0, ssem0, row, True)
                do_group(go + 1, 1, ssem1, row, True)

            drain_scatters(0, ssem0)
            drain_scatters(1, ssem1)
            plsc.subcore_barrier()
            pltpu.async_copy(acc_sh.at[pl.ds(t0, ROWS)],
                             agg.at[pl.ds(t0, ROWS), f], fsem).wait()
            plsc.subcore_barrier()

    return agg_kernel


EPT_D = E // (NSC * NTEC)
NB_D = EPT_D // B


def _deg_call():
    out_t = jax.ShapeDtypeStruct((NSC, SEG, 16), jnp.float32)
    scratch = [
        pltpu.VMEM((NB_D, B), jnp.int32),
        pltpu.VMEM((B, 16), jnp.float32),
        pltpu.VMEM((ZR, 16), jnp.float32),
        pltpu.VMEM_SHARED((SEG, 16), jnp.float32),
        pltpu.SemaphoreType.DMA,
        pltpu.SemaphoreType.DMA,
    ]

    @functools.partial(pl.kernel, out_type=out_t, mesh=_sc_mesh(),
                       scratch_types=scratch,
                       compiler_params=pltpu.CompilerParams(
                           use_tc_tiling_on_sc=False))
    def deg_kernel(combr, deg, comb_v, ones_v, zbuf, acc_sh, isem, fsem):
        cid = lax.axis_index("c")
        sid = lax.axis_index("s")
        tile = cid * NTEC + sid
        t0 = sid * ROWS
        pltpu.async_copy(combr.at[tile], comb_v, isem).wait()

        @pl.loop(0, B)
        def _fill_ones(i):
            ones_v[i] = jnp.full((16,), 1.0, jnp.float32)

        @pl.loop(0, ZR)
        def _fill_zeros(i):
            zbuf[i] = jnp.zeros((16,), jnp.float32)

        zc = [pltpu.async_copy(zbuf, acc_sh.at[pl.ds(t0 + ZR * z, ZR)], fsem)
              for z in range(NZ)]
        for c in zc:
            c.wait()
        plsc.subcore_barrier()

        @pl.loop(0, NB_D)
        def _blk(j):
            pltpu.sync_copy(ones_v, acc_sh.at[comb_v.at[j]], add=True)

        plsc.subcore_barrier()
        pltpu.async_copy(acc_sh.at[pl.ds(t0, ROWS)],
                         deg.at[cid, pl.ds(t0, ROWS)], fsem).wait()

    return deg_kernel


def _mm_kernel(h_ref, agg_ref, deg_ref, wrel_ref, wroot_ref, b_ref,
               g_ref, be_ref, m_ref, v_ref, o_ref, *, relu):
    acc = jnp.dot(h_ref[...], wroot_ref[...],
                  preferred_element_type=jnp.float32)
    deg = deg_ref[0] + deg_ref[1]
    for r in range(R):
        inv = 1.0 / jnp.maximum(deg[r][:, :1], 1.0)
        acc = acc + jnp.dot(agg_ref[r] * inv, wrel_ref[r],
                            preferred_element_type=jnp.float32)
    acc = acc + b_ref[...]
    acc = (acc - m_ref[...]) * (g_ref[...] * lax.rsqrt(v_ref[...] + 1e-5))
    acc = acc + be_ref[...]
    if relu:
        acc = jnp.maximum(acc, 0.0)
    o_ref[...] = acc


def _mm_call(h, agg, deg4, wrel, wroot, b, g, be, m, v, relu):
    din = h.shape[1]
    vspec = pl.BlockSpec((1, 512), lambda i: (0, 0))
    return pl.pallas_call(
        functools.partial(_mm_kernel, relu=relu),
        grid=(NBLK,),
        in_specs=[
            pl.BlockSpec((BN_BLK, din), lambda i: (i, 0)),
            pl.BlockSpec((R, BN_BLK, din), lambda i: (0, i, 0)),
            pl.BlockSpec((NSC, R, BN_BLK, 16), lambda i: (0, 0, i, 0)),
            pl.BlockSpec((R, din, 512), lambda i: (0, 0, 0)),
            pl.BlockSpec((din, 512), lambda i: (0, 0)),
            vspec, vspec, vspec, vspec, vspec,
        ],
        out_specs=pl.BlockSpec((BN_BLK, 512), lambda i: (i, 0)),
        out_shape=jax.ShapeDtypeStruct((N, 512), jnp.float32),
    )(h, agg, deg4, wrel, wroot, b, g, be, m, v)


def _pool_kernel(h_ref, batch_ref, w1_ref, b1_ref, w2_ref, b2_ref, o_ref,
                 sums_ref, cnt_ref):
    i = pl.program_id(0)

    @pl.when(i == 0)
    def _init():
        sums_ref[...] = jnp.zeros_like(sums_ref)
        cnt_ref[...] = jnp.zeros_like(cnt_ref)

    bb = batch_ref[0, 0, :]
    oh = (bb[:, None] == lax.broadcasted_iota(jnp.int32, (BN_BLK, NG), 1)
          ).astype(jnp.float32)
    dn = (((0,), (0,)), ((), ()))
    sums_ref[...] += lax.dot_general(oh, h_ref[...], dn,
                                     preferred_element_type=jnp.float32)
    cnt_ref[...] += lax.dot_general(oh, jnp.ones((BN_BLK, 8), jnp.float32),
                                    dn, preferred_element_type=jnp.float32)

    @pl.when(i == NBLK - 1)
    def _final():
        inv = 1.0 / jnp.maximum(cnt_ref[:, :1], 1.0)
        pooled = sums_ref[...] * inv
        hid = jnp.dot(pooled, w1_ref[...],
                      preferred_element_type=jnp.float32) + b1_ref[...]
        hid = jnp.maximum(hid, 0.0)
        o_ref[...] = jnp.dot(hid, w2_ref[...],
                             preferred_element_type=jnp.float32) + b2_ref[...]


def _pool_call(h, batch3, w1, b1, w2, b2):
    return pl.pallas_call(
        _pool_kernel,
        grid=(NBLK,),
        in_specs=[
            pl.BlockSpec((BN_BLK, 512), lambda i: (i, 0)),
            pl.BlockSpec((1, 1, BN_BLK), lambda i: (i, 0, 0)),
            pl.BlockSpec((512, 300), lambda i: (0, 0)),
            pl.BlockSpec((1, 300), lambda i: (0, 0)),
            pl.BlockSpec((300, 300), lambda i: (0, 0)),
            pl.BlockSpec((1, 300), lambda i: (0, 0)),
        ],
        out_specs=pl.BlockSpec((NG, 300), lambda i: (0, 0)),
        out_shape=jax.ShapeDtypeStruct((NG, 300), jnp.float32),
        scratch_shapes=[
            pltpu.VMEM((NG, 512), jnp.float32),
            pltpu.VMEM((NG, 8), jnp.float32),
        ],
    )(h, batch3, w1, b1, w2, b2)


def _scaled_src(src, fdim):
    f = jnp.arange(fdim, dtype=jnp.int32)[:, None]
    return (src[None, :] * fdim + f).reshape(fdim * NTEC, NGRP, K, B)


_agg32 = _make_agg_call(16)
_agg2 = _make_agg_call(1)
_deg = _deg_call()


def kernel(x, edge_index, edge_type, batch, W_rel1, W_root1, b1, W_rel2,
           W_root2, b2, W_rel3, W_root3, b3, W_rel4, W_root4, b4, W_rel5,
           W_root5, b5, W_rel6, W_root6, b6, bn_gamma, bn_beta, bn_mean,
           bn_var, ph_w1, ph_b1, ph_w2, ph_b2):
    src = edge_index[0]
    dst = edge_index[1]
    comb = edge_type * N + dst
    combr = comb.reshape(NTEC, NB, B)
    combd = comb.reshape(NSC * NTEC, NB_D, B)
    src32 = _scaled_src(src, 32)
    src2 = _scaled_src(src, 2)

    deg = _deg(combd)
    deg4 = deg.reshape(NSC, R, N, 16)

    g = bn_gamma.reshape(1, 512)
    be = bn_beta.reshape(1, 512)
    m = bn_mean.reshape(1, 512)
    v = bn_var.reshape(1, 512)

    x_pad = jnp.pad(x, ((0, 0), (0, 10)))
    w1_pad = jnp.pad(W_rel1, ((0, 0), (0, 10), (0, 0)))
    wr1_pad = jnp.pad(W_root1, ((0, 10), (0, 0)))

    layers = [
        (x_pad, src2, _agg2, 2, w1_pad, wr1_pad, b1),
        (None, src32, _agg32, 32, W_rel2, W_root2, b2),
        (None, src32, _agg32, 32, W_rel3, W_root3, b3),
        (None, src32, _agg32, 32, W_rel4, W_root4, b4),
        (None, src32, _agg32, 32, W_rel5, W_root5, b5),
        (None, src32, _agg32, 32, W_rel6, W_root6, b6),
    ]

    h = x_pad
    for li, (h0, srcall, aggf, fdim, wrel, wroot, bb) in enumerate(layers):
        h2 = h.reshape(N * fdim, 16)
        agg = aggf(h2, srcall, combr)
        agg_r = agg.reshape(R, N, fdim * 16)
        h = _mm_call(h, agg_r, deg4, wrel, wroot, bb.reshape(1, 512),
                     g, be, m, v, relu=(li < 5))

    return _pool_call(h, batch.reshape(NBLK, 1, BN_BLK), ph_w1,
                      ph_b1.reshape(1, 300), ph_w2, ph_b2.reshape(1, 300))


# speedup vs baseline: 2.0566x; 1.0100x over previous
import functools

import jax
import jax.numpy as jnp
from jax import lax
from jax.experimental import pallas as pl
from jax.experimental.pallas import tpu as pltpu
from jax.experimental.pallas import tpu_sc as plsc

N = 10000
E = 320000
R = 7
SEG = R * N
NG = 32

NTEC = 16
NSC = 2
EPT = E // NTEC
B = 80
NB = EPT // B
K = 5
NGRP = NB // K
ROWS = SEG // NTEC
ZR = 125
NZ = ROWS // ZR

BN_BLK = 400
NBLK = N // BN_BLK


def _sc_mesh():
    return plsc.VectorSubcoreMesh(core_axis_name="c", subcore_axis_name="s")


def _make_agg_call(nchunk_per_sc):
    fdim = nchunk_per_sc * NSC
    out_t = jax.ShapeDtypeStruct((SEG, fdim, 16), jnp.float32)
    scratch = [
        pltpu.VMEM((NB, B), jnp.int32),
        pltpu.VMEM((2, K, B), jnp.int32),
        pltpu.VMEM((2, K, B, 16), jnp.float32),
        pltpu.VMEM((ZR, 16), jnp.float32),
        pltpu.VMEM_SHARED((SEG, 16), jnp.float32),
        pltpu.SemaphoreType.DMA,
        pltpu.SemaphoreType.DMA,
        pltpu.SemaphoreType.DMA,
        pltpu.SemaphoreType.DMA,
        pltpu.SemaphoreType.DMA,
    ]

    @functools.partial(pl.kernel, out_type=out_t, mesh=_sc_mesh(),
                       scratch_types=scratch,
                       compiler_params=pltpu.CompilerParams(
                           use_tc_tiling_on_sc=False))
    def agg_kernel(h2, srcall, combr, agg, comb_v, sidx, gbuf, zbuf, acc_sh,
                   isem, gsem, ssem0, ssem1, fsem):
        cid = lax.axis_index("c")
        sid = lax.axis_index("s")
        t0 = sid * ROWS
        pltpu.async_copy(combr.at[sid], comb_v, isem).wait()

        @pl.loop(0, ZR)
        def _fill_zeros(i):
            zbuf[i] = jnp.zeros((16,), jnp.float32)

        def drain_scatters(s, ssem):
            for _ in range(K):
                pltpu.make_async_copy(gbuf.at[s, 0],
                                      acc_sh.at[pl.ds(0, B)], ssem).wait()

        def do_group(g, s, ssem, row, drain_prev):
            if drain_prev:
                drain_scatters(s, ssem)
            ic = [pltpu.async_copy(srcall.at[row, pl.ds((g * K + k) * B, B)],
                                   sidx.at[s, k], isem) for k in range(K)]
            for c in ic:
                c.wait()
            gc = [pltpu.async_copy(h2.at[sidx.at[s, k]], gbuf.at[s, k], gsem)
                  for k in range(K)]
            for c in gc:
                c.wait()
            for k in range(K):
                pltpu.async_copy(gbuf.at[s, k], acc_sh.at[comb_v.at[g * K + k]],
                                 ssem, add=True)

        @pl.loop(0, nchunk_per_sc)
        def _chunk(cc):
            f = cid * nchunk_per_sc + cc
            row = f * 16 + sid
            zc = [pltpu.async_copy(zbuf, acc_sh.at[pl.ds(t0 + ZR * z, ZR)],
                                   fsem) for z in range(NZ)]
            for c in zc:
                c.wait()
            plsc.subcore_barrier()
            do_group(0, 0, ssem0, row, False)
            do_group(1, 1, ssem1, row, False)

            @pl.loop(2, NGRP, step=2)
            def _grp(go):
                do_group(go, 0, ssem0, row, True)
                do_group(go + 1, 1, ssem1, row, True)

            drain_scatters(0, ssem0)
            drain_scatters(1, ssem1)
            plsc.subcore_barrier()
            pltpu.async_copy(acc_sh.at[pl.ds(t0, ROWS)],
                             agg.at[pl.ds(t0, ROWS), f], fsem).wait()
            plsc.subcore_barrier()

    return agg_kernel


EPT_D = E // (NSC * NTEC)
NB_D = EPT_D // B


def _deg_call():
    out_t = jax.ShapeDtypeStruct((NSC, SEG, 16), jnp.float32)
    scratch = [
        pltpu.VMEM((NB_D, B), jnp.int32),
        pltpu.VMEM((B, 16), jnp.float32),
        pltpu.VMEM((ZR, 16), jnp.float32),
        pltpu.VMEM_SHARED((SEG, 16), jnp.float32),
        pltpu.SemaphoreType.DMA,
        pltpu.SemaphoreType.DMA,
    ]

    @functools.partial(pl.kernel, out_type=out_t, mesh=_sc_mesh(),
                       scratch_types=scratch,
                       compiler_params=pltpu.CompilerParams(
                           use_tc_tiling_on_sc=False))
    def deg_kernel(combr, deg, comb_v, ones_v, zbuf, acc_sh, isem, fsem):
        cid = lax.axis_index("c")
        sid = lax.axis_index("s")
        tile = cid * NTEC + sid
        t0 = sid * ROWS
        pltpu.async_copy(combr.at[tile], comb_v, isem).wait()

        @pl.loop(0, B)
        def _fill_ones(i):
            ones_v[i] = jnp.full((16,), 1.0, jnp.float32)

        @pl.loop(0, ZR)
        def _fill_zeros(i):
            zbuf[i] = jnp.zeros((16,), jnp.float32)

        zc = [pltpu.async_copy(zbuf, acc_sh.at[pl.ds(t0 + ZR * z, ZR)], fsem)
              for z in range(NZ)]
        for c in zc:
            c.wait()
        plsc.subcore_barrier()

        @pl.loop(0, NB_D)
        def _blk(j):
            pltpu.sync_copy(ones_v, acc_sh.at[comb_v.at[j]], add=True)

        plsc.subcore_barrier()
        pltpu.async_copy(acc_sh.at[pl.ds(t0, ROWS)],
                         deg.at[cid, pl.ds(t0, ROWS)], fsem).wait()

    return deg_kernel


def _mm_kernel(h_ref, agg_ref, deg_ref, wrel_ref, wroot_ref, b_ref,
               g_ref, be_ref, m_ref, v_ref, o_ref, *, relu):
    acc = jnp.dot(h_ref[...], wroot_ref[...],
                  preferred_element_type=jnp.float32)
    deg = deg_ref[0] + deg_ref[1]
    for r in range(R):
        inv = 1.0 / jnp.maximum(deg[r][:, :1], 1.0)
        acc = acc + jnp.dot(agg_ref[r] * inv, wrel_ref[r],
                            preferred_element_type=jnp.float32)
    acc = acc + b_ref[...]
    acc = (acc - m_ref[...]) * (g_ref[...] * lax.rsqrt(v_ref[...] + 1e-5))
    acc = acc + be_ref[...]
    if relu:
        acc = jnp.maximum(acc, 0.0)
    o_ref[...] = acc


def _mm_call(h, agg, deg4, wrel, wroot, b, g, be, m, v, relu):
    din = h.shape[1]
    vspec = pl.BlockSpec((1, 512), lambda i: (0, 0))
    return pl.pallas_call(
        functools.partial(_mm_kernel, relu=relu),
        grid=(NBLK,),
        in_specs=[
            pl.BlockSpec((BN_BLK, din), lambda i: (i, 0)),
            pl.BlockSpec((R, BN_BLK, din), lambda i: (0, i, 0)),
            pl.BlockSpec((NSC, R, BN_BLK, 16), lambda i: (0, 0, i, 0)),
            pl.BlockSpec((R, din, 512), lambda i: (0, 0, 0)),
            pl.BlockSpec((din, 512), lambda i: (0, 0)),
            vspec, vspec, vspec, vspec, vspec,
        ],
        out_specs=pl.BlockSpec((BN_BLK, 512), lambda i: (i, 0)),
        out_shape=jax.ShapeDtypeStruct((N, 512), jnp.float32),
    )(h, agg, deg4, wrel, wroot, b, g, be, m, v)


def _pool_kernel(h_ref, batch_ref, w1_ref, b1_ref, w2_ref, b2_ref, o_ref,
                 sums_ref, cnt_ref):
    i = pl.program_id(0)

    @pl.when(i == 0)
    def _init():
        sums_ref[...] = jnp.zeros_like(sums_ref)
        cnt_ref[...] = jnp.zeros_like(cnt_ref)

    bb = batch_ref[0, 0, :]
    oh = (bb[:, None] == lax.broadcasted_iota(jnp.int32, (BN_BLK, NG), 1)
          ).astype(jnp.float32)
    dn = (((0,), (0,)), ((), ()))
    sums_ref[...] += lax.dot_general(oh, h_ref[...], dn,
                                     preferred_element_type=jnp.float32)
    cnt_ref[...] += lax.dot_general(oh, jnp.ones((BN_BLK, 8), jnp.float32),
                                    dn, preferred_element_type=jnp.float32)

    @pl.when(i == NBLK - 1)
    def _final():
        inv = 1.0 / jnp.maximum(cnt_ref[:, :1], 1.0)
        pooled = sums_ref[...] * inv
        hid = jnp.dot(pooled, w1_ref[...],
                      preferred_element_type=jnp.float32) + b1_ref[...]
        hid = jnp.maximum(hid, 0.0)
        o_ref[...] = jnp.dot(hid, w2_ref[...],
                             preferred_element_type=jnp.float32) + b2_ref[...]


def _pool_call(h, batch3, w1, b1, w2, b2):
    return pl.pallas_call(
        _pool_kernel,
        grid=(NBLK,),
        in_specs=[
            pl.BlockSpec((BN_BLK, 512), lambda i: (i, 0)),
            pl.BlockSpec((1, 1, BN_BLK), lambda i: (i, 0, 0)),
            pl.BlockSpec((512, 300), lambda i: (0, 0)),
            pl.BlockSpec((1, 300), lambda i: (0, 0)),
            pl.BlockSpec((300, 300), lambda i: (0, 0)),
            pl.BlockSpec((1, 300), lambda i: (0, 0)),
        ],
        out_specs=pl.BlockSpec((NG, 300), lambda i: (0, 0)),
        out_shape=jax.ShapeDtypeStruct((NG, 300), jnp.float32),
        scratch_shapes=[
            pltpu.VMEM((NG, 512), jnp.float32),
            pltpu.VMEM((NG, 8), jnp.float32),
        ],
    )(h, batch3, w1, b1, w2, b2)


def _scaled_src(src, fdim):
    f = jnp.arange(fdim, dtype=jnp.int32)[:, None]
    return (src[None, :] * fdim + f).reshape(fdim * NTEC, EPT)


_agg32 = _make_agg_call(16)
_agg2 = _make_agg_call(1)
_deg = _deg_call()


def kernel(x, edge_index, edge_type, batch, W_rel1, W_root1, b1, W_rel2,
           W_root2, b2, W_rel3, W_root3, b3, W_rel4, W_root4, b4, W_rel5,
           W_root5, b5, W_rel6, W_root6, b6, bn_gamma, bn_beta, bn_mean,
           bn_var, ph_w1, ph_b1, ph_w2, ph_b2):
    src = edge_index[0]
    dst = edge_index[1]
    comb = edge_type * N + dst
    combr = comb.reshape(NTEC, NB, B)
    combd = comb.reshape(NSC * NTEC, NB_D, B)
    src32 = _scaled_src(src, 32)
    src2 = _scaled_src(src, 2)

    deg = _deg(combd)
    deg4 = deg.reshape(NSC, R, N, 16)

    g = bn_gamma.reshape(1, 512)
    be = bn_beta.reshape(1, 512)
    m = bn_mean.reshape(1, 512)
    v = bn_var.reshape(1, 512)

    x_pad = jnp.pad(x, ((0, 0), (0, 10)))
    w1_pad = jnp.pad(W_rel1, ((0, 0), (0, 10), (0, 0)))
    wr1_pad = jnp.pad(W_root1, ((0, 10), (0, 0)))

    layers = [
        (x_pad, src2, _agg2, 2, w1_pad, wr1_pad, b1),
        (None, src32, _agg32, 32, W_rel2, W_root2, b2),
        (None, src32, _agg32, 32, W_rel3, W_root3, b3),
        (None, src32, _agg32, 32, W_rel4, W_root4, b4),
        (None, src32, _agg32, 32, W_rel5, W_root5, b5),
        (None, src32, _agg32, 32, W_rel6, W_root6, b6),
    ]

    h = x_pad
    for li, (h0, srcall, aggf, fdim, wrel, wroot, bb) in enumerate(layers):
        h2 = h.reshape(N * fdim, 16)
        agg = aggf(h2, srcall, combr)
        agg_r = agg.reshape(R, N, fdim * 16)
        h = _mm_call(h, agg_r, deg4, wrel, wroot, bb.reshape(1, 512),
                     g, be, m, v, relu=(li < 5))

    return _pool_call(h, batch.reshape(NBLK, 1, BN_BLK), ph_w1,
                      ph_b1.reshape(1, 300), ph_w2, ph_b2.reshape(1, 300))
